# baseline (device time: 28646 ns/iter reference)
import jax
import jax.numpy as jnp
from jax import lax
from jax.experimental import pallas as pl
from jax.experimental.pallas import tpu as pltpu

N_DEV = 4
B_LOC = 2
SQ = 256
SKV = 256
HQ = 4
DH = 64
D_MODEL = 512
D_CHUNK = HQ * DH


def _body(x_ref, wq_ref, wo_ref, k_ref, v_ref, out_ref,
          commq, commo, sendq, sendo, recvq, recvo):
    my = lax.axis_index("i")
    right = (my + 1) % N_DEV
    left = (my + N_DEV - 1) % N_DEV
    diag = (my + 2) % N_DEV

    barrier = pltpu.get_barrier_semaphore()
    for nbr in (left, right, diag):
        pl.semaphore_signal(barrier, inc=1, device_id=(nbr,),
                            device_id_type=pl.DeviceIdType.MESH)
    pl.semaphore_wait(barrier, 3)

    sends = []
    for slot, tgt in ((0, right), (1, left), (2, diag)):
        for src, comm, ssem, rsem in ((wq_ref, commq, sendq, recvq),
                                      (wo_ref, commo, sendo, recvo)):
            rdma = pltpu.make_async_remote_copy(
                src_ref=src,
                dst_ref=comm.at[slot],
                send_sem=ssem.at[slot],
                recv_sem=rsem.at[slot],
                device_id=(tgt,),
                device_id_type=pl.DeviceIdType.MESH,
            )
            rdma.start()
            sends.append(rdma)

    qi = lax.broadcasted_iota(jnp.int32, (SQ, SKV), 0)
    ki = lax.broadcasted_iota(jnp.int32, (SQ, SKV), 1)
    mask = (jnp.abs(qi - ki) <= 128) | (ki < 32) | (qi < 32)
    bias = jnp.where(mask, 0.0, -1e9).astype(jnp.float32)

    def do_phase(p, wq, wo):
        q = lax.dot_general(x_ref[...], wq, (((1,), (0,)), ((), ())),
                            preferred_element_type=jnp.float32)
        qb = (q * 0.125).astype(jnp.bfloat16)
        for b in range(B_LOC):
            ctx_parts = []
            for hh in range(HQ):
                idx = (p * HQ + hh) * B_LOC + b
                qs = qb[b * SQ:(b + 1) * SQ, hh * DH:(hh + 1) * DH]
                k = k_ref[idx]
                s = lax.dot_general(qs, k, (((1,), (1,)), ((), ())),
                                    preferred_element_type=jnp.float32)
                s = s + bias
                m = jnp.max(s, axis=1, keepdims=True)
                w = jnp.exp(s - m)
                den = jnp.sum(w, axis=1, keepdims=True)
                wb = (w / den).astype(jnp.bfloat16)
                v = v_ref[idx]
                ctx = lax.dot_general(wb, v, (((1,), (0,)), ((), ())),
                                      preferred_element_type=jnp.float32)
                ctx_parts.append(ctx.astype(jnp.bfloat16))
            ctxb = jnp.concatenate(ctx_parts, axis=1)
            o = lax.dot_general(ctxb, wo, (((1,), (0,)), ((), ())),
                                preferred_element_type=jnp.float32)
            if p == 0:
                out_ref[b * SQ:(b + 1) * SQ, :] = o
            else:
                out_ref[b * SQ:(b + 1) * SQ, :] += o

    do_phase(0, wq_ref[...], wo_ref[...])

    for p in range(1, N_DEV):
        slot = p - 1
        for comm, ssem, rsem in ((commq, sendq, recvq), (commo, sendo, recvo)):
            recv = pltpu.make_async_remote_copy(
                src_ref=wq_ref if comm is commq else wo_ref,
                dst_ref=comm.at[slot],
                send_sem=ssem.at[slot],
                recv_sem=rsem.at[slot],
                device_id=(left,),
                device_id_type=pl.DeviceIdType.MESH,
            )
            recv.wait_recv()
        do_phase(p, commq[slot], commo[slot])

    for rdma in sends:
        rdma.wait_send()


def kernel(x, Wq, K_ext, V_ext, Wo):
    my = lax.axis_index("i")

    xb = x.astype(jnp.bfloat16).reshape(B_LOC * SQ, D_MODEL)
    wqb = Wq.astype(jnp.bfloat16)
    wob = Wo.astype(jnp.bfloat16)

    order = jnp.mod(my + jnp.array([0, -1, 1, 2]), N_DEV)

    def prep(t):
        t = lax.dynamic_slice_in_dim(t, my * B_LOC, B_LOC, 0)
        t = t.astype(jnp.bfloat16).transpose(2, 0, 1, 3)
        t = t.reshape(N_DEV, HQ, B_LOC, SKV, DH)
        t = jnp.take(t, order, axis=0)
        return t.reshape(N_DEV * HQ * B_LOC, SKV, DH)

    kp = prep(K_ext)
    vp = prep(V_ext)

    out = pl.pallas_call(
        _body,
        out_shape=jax.ShapeDtypeStruct((B_LOC * SQ, D_MODEL), jnp.float32),
        in_specs=[pl.BlockSpec(memory_space=pltpu.VMEM)] * 5,
        out_specs=pl.BlockSpec(memory_space=pltpu.VMEM),
        scratch_shapes=[
            pltpu.VMEM((3, D_MODEL, D_CHUNK), jnp.bfloat16),
            pltpu.VMEM((3, D_CHUNK, D_MODEL), jnp.bfloat16),
            pltpu.SemaphoreType.DMA((3,)),
            pltpu.SemaphoreType.DMA((3,)),
            pltpu.SemaphoreType.DMA((3,)),
            pltpu.SemaphoreType.DMA((3,)),
        ],
        compiler_params=pltpu.CompilerParams(collective_id=0),
    )(xb, wqb, wob, kp, vp)
    return out.reshape(B_LOC, SQ, D_MODEL)
